# baseline (speedup 1.0000x reference)
import functools
import jax
import jax.numpy as jnp
from jax import lax
from jax.experimental import pallas as pl
from jax.experimental.pallas import tpu as pltpu
from jax.experimental.pallas import tpu_sc as plsc

_NUM_USERS = 1_000_000
_NUM_SUBGROUPS = 1_000
_NUM_CHOICES = 100_000
_D = 64

_U_TC = 500_000
_U_SC = _NUM_USERS - _U_TC

_SCR_ROWS = 8_000
_N_TC = _U_TC // _SCR_ROWS
_TC_REM = _U_TC - _N_TC * _SCR_ROWS


def _tc_kernel(user_ref, full_in, user_out, scratch, sem_u):
    del full_in
    scratch[...] = jnp.broadcast_to(user_ref[...], (_SCR_ROWS, _D))
    copies = []
    for i in range(_N_TC):
        c = pltpu.make_async_copy(
            scratch,
            user_out.at[pl.ds(i * _SCR_ROWS, _SCR_ROWS), :],
            sem_u)
        c.start()
        copies.append(c)
    rem = pltpu.make_async_copy(
        scratch.at[pl.ds(0, _TC_REM), :],
        user_out.at[pl.ds(_N_TC * _SCR_ROWS, _TC_REM), :],
        sem_u)
    rem.start()
    for c in copies:
        c.wait()
    rem.wait()


_NW = 32

_BUF = 800
_N_CHUNKS = _U_SC // _BUF
_U_ROUNDS = _N_CHUNKS // _NW
_U_TAIL = _N_CHUNKS - _U_ROUNDS * _NW

_CBUF = 200
_NC_CHUNKS = _NUM_CHOICES // _CBUF
_C_ROUNDS = _NC_CHUNKS // _NW
_C_TAIL = _NC_CHUNKS - _C_ROUNDS * _NW

_SUB_WORKERS = 5
_SPW = _NUM_SUBGROUPS // _SUB_WORKERS


def _sc_body(sub_hbm, cho_hbm, user_hbm,
             sub_out, cho_out, user_out,
             buf, cbuf, sem_u, sem_c, sem_s):
    wid = lax.axis_index("c") * 16 + lax.axis_index("s")

    pltpu.sync_copy(user_hbm, buf.at[pl.ds(0, 1), :])
    regs = [buf[0, pl.ds(k * 16, 16)] for k in range(_D // 16)]

    def _fill_row(j, carry):
        for k in range(_D // 16):
            buf[j, pl.ds(k * 16, 16)] = regs[k]
        return carry

    lax.fori_loop(1, _BUF, _fill_row, 0)

    user_copies = []
    for j in range(_U_ROUNDS):
        c = pltpu.make_async_copy(
            buf,
            user_out.at[pl.ds(_U_TC + (j * _NW + wid) * _BUF, _BUF), :],
            sem_u)
        c.start()
        user_copies.append(c)

    u_tail = pltpu.make_async_copy(
        buf,
        user_out.at[pl.ds(_U_TC + (_U_ROUNDS * _NW + wid) * _BUF, _BUF), :],
        sem_u)

    @pl.when(wid < _U_TAIL)
    def _():
        u_tail.start()

    for r in range(_C_ROUNDS):
        base = (r * _NW + wid) * _CBUF
        rd = pltpu.make_async_copy(
            cho_hbm.at[pl.ds(base, _CBUF), :], cbuf, sem_c)
        wr = pltpu.make_async_copy(
            cbuf, cho_out.at[pl.ds(base, _CBUF), :], sem_c)
        rd.start()
        rd.wait()
        wr.start()
        wr.wait()

    tbase = (_C_ROUNDS * _NW + wid) * _CBUF
    t_rd = pltpu.make_async_copy(
        cho_hbm.at[pl.ds(tbase, _CBUF), :], cbuf, sem_c)
    t_wr = pltpu.make_async_copy(
        cbuf, cho_out.at[pl.ds(tbase, _CBUF), :], sem_c)

    @pl.when(wid < _C_TAIL)
    def _():
        t_rd.start()
        t_rd.wait()
        t_wr.start()
        t_wr.wait()

    sbase = wid * _SPW
    sub_read = pltpu.make_async_copy(
        sub_hbm.at[pl.ds(sbase, _SPW), :], cbuf, sem_s)
    sub_write = pltpu.make_async_copy(
        cbuf, sub_out.at[pl.ds(sbase, _SPW), :], sem_s)

    @pl.when(wid < _SUB_WORKERS)
    def _():
        sub_read.start()
        sub_read.wait()
        sub_write.start()
        sub_write.wait()

    for c in user_copies:
        c.wait()

    @pl.when(wid < _U_TAIL)
    def _():
        u_tail.wait()


@functools.partial(
    pl.kernel,
    out_type=[
        jax.ShapeDtypeStruct((_NUM_SUBGROUPS, _D), jnp.float32),
        jax.ShapeDtypeStruct((_NUM_CHOICES, _D), jnp.float32),
        jax.ShapeDtypeStruct((_NUM_USERS, _D), jnp.float32),
    ],
    mesh=plsc.VectorSubcoreMesh(core_axis_name="c", subcore_axis_name="s"),
    scratch_types=[
        pltpu.VMEM((_BUF, _D), jnp.float32),
        pltpu.VMEM((_CBUF, _D), jnp.float32),
        pltpu.SemaphoreType.DMA,
        pltpu.SemaphoreType.DMA,
        pltpu.SemaphoreType.DMA,
    ],
)
def _sc_tables(sub_hbm, cho_hbm, user_hbm, sub_out, cho_out, user_out,
               buf, cbuf, sem_u, sem_c, sem_s):
    _sc_body(sub_hbm, cho_hbm, user_hbm, sub_out, cho_out, user_out,
             buf, cbuf, sem_u, sem_c, sem_s)


def kernel(sub_w, cho_w, user_token):
    sub_o, cho_o, user_half = _sc_tables(sub_w, cho_w, user_token)

    user_o = pl.pallas_call(
        _tc_kernel,
        in_specs=[
            pl.BlockSpec(memory_space=pltpu.MemorySpace.VMEM),
            pl.BlockSpec(memory_space=pltpu.MemorySpace.HBM),
        ],
        out_specs=pl.BlockSpec(memory_space=pltpu.MemorySpace.HBM),
        out_shape=jax.ShapeDtypeStruct((_NUM_USERS, _D), jnp.float32),
        scratch_shapes=[
            pltpu.VMEM((_SCR_ROWS, _D), jnp.float32),
            pltpu.SemaphoreType.DMA,
        ],
        input_output_aliases={1: 0},
    )(user_token, user_half)

    return (sub_o, cho_o, user_o)


# speedup vs baseline: 1.0160x; 1.0160x over previous
import functools
import jax
import jax.numpy as jnp
from jax import lax
from jax.experimental import pallas as pl
from jax.experimental.pallas import tpu as pltpu
from jax.experimental.pallas import tpu_sc as plsc

_NUM_USERS = 1_000_000
_NUM_SUBGROUPS = 1_000
_NUM_CHOICES = 100_000
_D = 64

_NW = 32

_BUF = 400
_N_CHUNKS = _NUM_USERS // _BUF
_U_ROUNDS = _N_CHUNKS // _NW
_U_TAIL = _N_CHUNKS - _U_ROUNDS * _NW

_CBUF = 200
_NC_CHUNKS = _NUM_CHOICES // _CBUF
_C_ROUNDS = _NC_CHUNKS // _NW
_C_TAIL = _NC_CHUNKS - _C_ROUNDS * _NW

_SUB_WORKERS = 5
_SPW = _NUM_SUBGROUPS // _SUB_WORKERS


def _sc_body(sub_hbm, cho_hbm, user_hbm,
             sub_out, cho_out, user_out,
             buf, cbuf0, cbuf1, sem_c0, sem_c1, sem_u, sem_s):
    wid = lax.axis_index("c") * 16 + lax.axis_index("s")

    cbufs = [cbuf0, cbuf1]
    csems = [sem_c0, sem_c1]
    pending = [None, None]
    for r in range(_C_ROUNDS):
        base = (r * _NW + wid) * _CBUF
        p = r % 2
        if pending[p] is not None:
            pending[p].wait()
        rd = pltpu.make_async_copy(
            cho_hbm.at[pl.ds(base, _CBUF), :], cbufs[p], csems[p])
        rd.start()
        rd.wait()
        wr = pltpu.make_async_copy(
            cbufs[p], cho_out.at[pl.ds(base, _CBUF), :], csems[p])
        wr.start()
        pending[p] = wr

    tbase = (_C_ROUNDS * _NW + wid) * _CBUF
    p = _C_ROUNDS % 2
    t_rd = pltpu.make_async_copy(
        cho_hbm.at[pl.ds(tbase, _CBUF), :], cbufs[p], csems[p])
    t_wr = pltpu.make_async_copy(
        cbufs[p], cho_out.at[pl.ds(tbase, _CBUF), :], csems[p])

    @pl.when(wid < _C_TAIL)
    def _():
        if pending[p] is not None:
            pending[p].wait()
        t_rd.start()
        t_rd.wait()
        t_wr.start()
        t_wr.wait()

    @pl.when(wid >= _C_TAIL)
    def _():
        if pending[p] is not None:
            pending[p].wait()

    if pending[1 - p] is not None:
        pending[1 - p].wait()

    sbase = wid * _SPW
    sub_read = pltpu.make_async_copy(
        sub_hbm.at[pl.ds(sbase, _SPW), :], cbuf0, sem_s)
    sub_write = pltpu.make_async_copy(
        cbuf0, sub_out.at[pl.ds(sbase, _SPW), :], sem_s)

    @pl.when(wid < _SUB_WORKERS)
    def _():
        sub_read.start()
        sub_read.wait()
        sub_write.start()
        sub_write.wait()

    pltpu.sync_copy(user_hbm, buf.at[pl.ds(0, 1), :])
    regs = [buf[0, pl.ds(k * 16, 16)] for k in range(_D // 16)]

    def _fill_row(j, carry):
        for k in range(_D // 16):
            buf[j, pl.ds(k * 16, 16)] = regs[k]
        return carry

    lax.fori_loop(1, _BUF, _fill_row, 0)

    user_copies = []
    for j in range(_U_ROUNDS):
        c = pltpu.make_async_copy(
            buf,
            user_out.at[pl.ds((j * _NW + wid) * _BUF, _BUF), :],
            sem_u)
        c.start()
        user_copies.append(c)

    u_tail = pltpu.make_async_copy(
        buf,
        user_out.at[pl.ds((_U_ROUNDS * _NW + wid) * _BUF, _BUF), :],
        sem_u)

    @pl.when(wid < _U_TAIL)
    def _():
        u_tail.start()

    for c in user_copies:
        c.wait()

    @pl.when(wid < _U_TAIL)
    def _():
        u_tail.wait()


@functools.partial(
    pl.kernel,
    out_type=[
        jax.ShapeDtypeStruct((_NUM_SUBGROUPS, _D), jnp.float32),
        jax.ShapeDtypeStruct((_NUM_CHOICES, _D), jnp.float32),
        jax.ShapeDtypeStruct((_NUM_USERS, _D), jnp.float32),
    ],
    mesh=plsc.VectorSubcoreMesh(core_axis_name="c", subcore_axis_name="s"),
    scratch_types=[
        pltpu.VMEM((_BUF, _D), jnp.float32),
        pltpu.VMEM((_CBUF, _D), jnp.float32),
        pltpu.VMEM((_CBUF, _D), jnp.float32),
        pltpu.SemaphoreType.DMA,
        pltpu.SemaphoreType.DMA,
        pltpu.SemaphoreType.DMA,
        pltpu.SemaphoreType.DMA,
    ],
)
def _sc_tables(sub_hbm, cho_hbm, user_hbm, sub_out, cho_out, user_out,
               buf, cbuf0, cbuf1, sem_c0, sem_c1, sem_u, sem_s):
    _sc_body(sub_hbm, cho_hbm, user_hbm, sub_out, cho_out, user_out,
             buf, cbuf0, cbuf1, sem_c0, sem_c1, sem_u, sem_s)


def kernel(sub_w, cho_w, user_token):
    sub_o, cho_o, user_o = _sc_tables(sub_w, cho_w, user_token)
    return (sub_o, cho_o, user_o)
